# baseline (device time: 184792 ns/iter reference)
import jax
import jax.numpy as jnp
from jax import lax
from jax.experimental import pallas as pl
from jax.experimental.pallas import tpu as pltpu

N_DEV = 8
B = 1024
N_OUT = 4096
QN = N_OUT // 4
SB = B // 4


def kernel(x, w_mat):
    m_glob, k_sh = x.shape
    k_glob, n = w_mat.shape
    assert k_sh == B and m_glob == N_DEV * B and n == N_OUT

    def body(x_hbm, w_hbm, out_ref, x16_hbm, recv_vmem, w_vmem, c16_vmem,
             wb16_vmem, send_sems, recv_sems, last_send_sems,
             last_recv_sems, w_sems, co_sems, loc_sem):
        me = lax.axis_index("i")

        barrier_sem = pltpu.get_barrier_semaphore()
        for k in range(1, N_DEV):
            pl.semaphore_signal(
                barrier_sem, inc=1,
                device_id=((me + k) % N_DEV,),
                device_id_type=pl.DeviceIdType.MESH,
            )

        def cast_in(k):
            dst = (me - k) % N_DEV
            return pltpu.make_async_copy(
                x_hbm.at[pl.ds(dst * B, B), :],
                w_vmem.at[k % 2],
                w_sems.at[k % 2],
            )

        loc_cp = pltpu.make_async_copy(
            x_hbm.at[pl.ds(me * B, B), :], w_vmem.at[0], loc_sem)
        loc_cp.start()
        cast_in(1).start()
        loc_cp.wait()
        recv_vmem[0] = w_vmem[0].astype(jnp.bfloat16)

        pl.semaphore_wait(barrier_sem, N_DEV - 1)

        rdmas = []
        last_rdmas = []
        for k in range(1, N_DEV):
            buf = k % 2
            dst = (me - k) % N_DEV
            if k + 1 < N_DEV:
                cast_in(k + 1).start()
            cast_in(k).wait()
            c16_vmem[buf] = w_vmem[buf].astype(jnp.bfloat16)
            co = pltpu.make_async_copy(
                c16_vmem.at[buf], x16_hbm.at[pl.ds(dst * B, B), :],
                co_sems.at[buf])
            co.start()
            co.wait()
            if k < N_DEV - 1:
                rdma = pltpu.make_async_remote_copy(
                    src_ref=x16_hbm.at[pl.ds(dst * B, B), :],
                    dst_ref=recv_vmem.at[k],
                    send_sem=send_sems.at[k],
                    recv_sem=recv_sems.at[k],
                    device_id=(dst,),
                    device_id_type=pl.DeviceIdType.MESH,
                )
                rdma.start()
                rdmas.append(rdma)
            else:
                for j in range(4):
                    sub = pltpu.make_async_remote_copy(
                        src_ref=x16_hbm.at[pl.ds(dst * B + j * SB, SB), :],
                        dst_ref=recv_vmem.at[k, pl.ds(j * SB, SB), :],
                        send_sem=last_send_sems.at[j],
                        recv_sem=last_recv_sems.at[j],
                        device_id=(dst,),
                        device_id_type=pl.DeviceIdType.MESH,
                    )
                    sub.start()
                    last_rdmas.append(sub)

        def w_copy(u):
            t, q = divmod(u, 4)
            s = (me + t) % N_DEV
            return pltpu.make_async_copy(
                w_hbm.at[pl.ds(s * B, B), pl.ds(q * QN, QN)],
                w_vmem.at[u % 2],
                w_sems.at[u % 2],
            )

        c = 0.7978845608028654

        def gelu(y):
            return 0.5 * y * (1.0 + jnp.tanh(c * (y + 0.044715 * y * y * y)))

        w_copy(0).start()
        for u in range(4 * (N_DEV - 1)):
            t, q = divmod(u, 4)
            if u + 1 <= 4 * (N_DEV - 1):
                w_copy(u + 1).start()
            if q == 0 and t > 0:
                rdmas[t - 1].wait_recv()
            w_copy(u).wait()
            part = lax.dot_general(
                recv_vmem[t],
                w_vmem[u % 2].astype(jnp.bfloat16),
                (((1,), (0,)), ((), ())),
                preferred_element_type=jnp.float32,
            )
            if t == 0:
                out_ref[:, q * QN:(q + 1) * QN] = part
            else:
                out_ref[:, q * QN:(q + 1) * QN] += part

        for q in range(4):
            u = 4 * (N_DEV - 1) + q
            if q < 3:
                w_copy(u + 1).start()
            w_copy(u).wait()
            wb16_vmem[q] = w_vmem[u % 2].astype(jnp.bfloat16)
        for j in range(4):
            last_rdmas[j].wait_recv()
            xs = recv_vmem[N_DEV - 1, j * SB:(j + 1) * SB, :]
            rows = slice(j * SB, (j + 1) * SB)
            for q in range(4):
                p = lax.dot_general(
                    xs, wb16_vmem[q], (((1,), (0,)), ((), ())),
                    preferred_element_type=jnp.float32)
                sl = slice(q * QN, (q + 1) * QN)
                out_ref[rows, sl] = gelu(out_ref[rows, sl] + p)

        for rdma in rdmas + last_rdmas:
            rdma.wait_send()

    out, _x16 = pl.pallas_call(
        body,
        out_shape=[
            jax.ShapeDtypeStruct((B, N_OUT), jnp.float32),
            jax.ShapeDtypeStruct((N_DEV * B, B), jnp.bfloat16),
        ],
        in_specs=[
            pl.BlockSpec(memory_space=pltpu.HBM),
            pl.BlockSpec(memory_space=pltpu.HBM),
        ],
        out_specs=[
            pl.BlockSpec(memory_space=pltpu.VMEM),
            pl.BlockSpec(memory_space=pltpu.HBM),
        ],
        scratch_shapes=[
            pltpu.VMEM((N_DEV, B, B), jnp.bfloat16),
            pltpu.VMEM((2, B, QN), jnp.float32),
            pltpu.VMEM((2, B, B), jnp.bfloat16),
            pltpu.VMEM((4, B, QN), jnp.bfloat16),
            pltpu.SemaphoreType.DMA((N_DEV,)),
            pltpu.SemaphoreType.DMA((N_DEV,)),
            pltpu.SemaphoreType.DMA((4,)),
            pltpu.SemaphoreType.DMA((4,)),
            pltpu.SemaphoreType.DMA((2,)),
            pltpu.SemaphoreType.DMA((2,)),
            pltpu.SemaphoreType.DMA,
        ],
        compiler_params=pltpu.CompilerParams(
            collective_id=0,
            vmem_limit_bytes=64 * 1024 * 1024,
        ),
    )(x, w_mat)
    return out


# device time: 179693 ns/iter; 1.0284x vs baseline; 1.0284x over previous
import jax
import jax.numpy as jnp
from jax import lax
from jax.experimental import pallas as pl
from jax.experimental.pallas import tpu as pltpu

N_DEV = 8
B = 1024
N_OUT = 4096
HN = N_OUT // 2


def kernel(x, w_mat):
    m_glob, k_sh = x.shape
    k_glob, n = w_mat.shape
    assert k_sh == B and m_glob == N_DEV * B and n == N_OUT

    def body(x_hbm, w_hbm, out_ref, x16_hbm, recv_vmem, w_vmem, c16_vmem,
             send_sems, recv_sems, w_sems, co_sems, loc_sem):
        me = lax.axis_index("i")

        barrier_sem = pltpu.get_barrier_semaphore()
        for k in range(1, N_DEV):
            pl.semaphore_signal(
                barrier_sem, inc=1,
                device_id=((me + k) % N_DEV,),
                device_id_type=pl.DeviceIdType.MESH,
            )

        def cast_in(k):
            dst = (me - k) % N_DEV
            return pltpu.make_async_copy(
                x_hbm.at[pl.ds(dst * B, B), :],
                w_vmem.at[k % 2, :, pl.ds(0, B)],
                w_sems.at[k % 2],
            )

        loc_cp = pltpu.make_async_copy(
            x_hbm.at[pl.ds(me * B, B), :],
            w_vmem.at[0, :, pl.ds(0, B)], loc_sem)
        loc_cp.start()
        cast_in(1).start()
        loc_cp.wait()
        recv_vmem[0] = w_vmem[0, :, :B].astype(jnp.bfloat16)

        pl.semaphore_wait(barrier_sem, N_DEV - 1)

        rdmas = []
        for k in range(1, N_DEV):
            buf = k % 2
            dst = (me - k) % N_DEV
            if k + 1 < N_DEV:
                cast_in(k + 1).start()
            cast_in(k).wait()
            c16_vmem[buf] = w_vmem[buf, :, :B].astype(jnp.bfloat16)
            co = pltpu.make_async_copy(
                c16_vmem.at[buf], x16_hbm.at[pl.ds(dst * B, B), :],
                co_sems.at[buf])
            co.start()
            co.wait()
            rdma = pltpu.make_async_remote_copy(
                src_ref=x16_hbm.at[pl.ds(dst * B, B), :],
                dst_ref=recv_vmem.at[k],
                send_sem=send_sems.at[k],
                recv_sem=recv_sems.at[k],
                device_id=(dst,),
                device_id_type=pl.DeviceIdType.MESH,
            )
            rdma.start()
            rdmas.append(rdma)

        def w_copy(u):
            t, h = divmod(u, 2)
            s = (me + t) % N_DEV
            return pltpu.make_async_copy(
                w_hbm.at[pl.ds(s * B, B), pl.ds(h * HN, HN)],
                w_vmem.at[u % 2],
                w_sems.at[u % 2],
            )

        c = 0.7978845608028654

        def gelu(y):
            return 0.5 * y * (1.0 + jnp.tanh(c * (y + 0.044715 * y * y * y)))

        w_copy(0).start()
        for u in range(2 * N_DEV):
            t, h = divmod(u, 2)
            if u + 1 < 2 * N_DEV:
                w_copy(u + 1).start()
            if h == 0 and t > 0:
                rdmas[t - 1].wait_recv()
            w_copy(u).wait()
            part = lax.dot_general(
                recv_vmem[t],
                w_vmem[u % 2].astype(jnp.bfloat16),
                (((1,), (0,)), ((), ())),
                preferred_element_type=jnp.float32,
            )
            if t == 0:
                out_ref[:, h * HN:(h + 1) * HN] = part
            elif t < N_DEV - 1:
                out_ref[:, h * HN:(h + 1) * HN] += part
            else:
                for g in range(2):
                    lo = g * (HN // 2)
                    sl = slice(h * HN + lo, h * HN + lo + HN // 2)
                    out_ref[:, sl] = gelu(
                        out_ref[:, sl] + part[:, lo:lo + HN // 2])

        for rdma in rdmas:
            rdma.wait_send()

    out, _x16 = pl.pallas_call(
        body,
        out_shape=[
            jax.ShapeDtypeStruct((B, N_OUT), jnp.float32),
            jax.ShapeDtypeStruct((N_DEV * B, B), jnp.bfloat16),
        ],
        in_specs=[
            pl.BlockSpec(memory_space=pltpu.HBM),
            pl.BlockSpec(memory_space=pltpu.HBM),
        ],
        out_specs=[
            pl.BlockSpec(memory_space=pltpu.VMEM),
            pl.BlockSpec(memory_space=pltpu.HBM),
        ],
        scratch_shapes=[
            pltpu.VMEM((N_DEV, B, B), jnp.bfloat16),
            pltpu.VMEM((2, B, HN), jnp.float32),
            pltpu.VMEM((2, B, B), jnp.bfloat16),
            pltpu.SemaphoreType.DMA((N_DEV,)),
            pltpu.SemaphoreType.DMA((N_DEV,)),
            pltpu.SemaphoreType.DMA((2,)),
            pltpu.SemaphoreType.DMA((2,)),
            pltpu.SemaphoreType.DMA,
        ],
        compiler_params=pltpu.CompilerParams(
            collective_id=0,
            vmem_limit_bytes=64 * 1024 * 1024,
        ),
    )(x, w_mat)
    return out
